# baseline (device time: 71856 ns/iter reference)
import jax
import jax.numpy as jnp
from jax import lax
from jax.experimental import pallas as pl
from jax.experimental.pallas import tpu as pltpu


def kernel(dy, W):
    m, k = dy.shape
    n, k2 = W.shape
    assert k == k2

    def body(dy_ref, w_ref, out_ref, comm_ref, send_sem, recv_sem):
        my_x = lax.axis_index("x")
        my_y = lax.axis_index("y")
        my_z = lax.axis_index("z")
        partner = (my_x, 1 - my_y, my_z)

        barrier_sem = pltpu.get_barrier_semaphore()
        pl.semaphore_signal(
            barrier_sem, inc=1,
            device_id=partner, device_id_type=pl.DeviceIdType.MESH,
        )
        pl.semaphore_wait(barrier_sem, 1)

        out_ref[...] = lax.dot_general(
            dy_ref[...], w_ref[...],
            dimension_numbers=(((1,), (1,)), ((), ())),
            preferred_element_type=jnp.float32,
        )

        rdma = pltpu.make_async_remote_copy(
            src_ref=out_ref,
            dst_ref=comm_ref,
            send_sem=send_sem,
            recv_sem=recv_sem,
            device_id=partner,
            device_id_type=pl.DeviceIdType.MESH,
        )
        rdma.start()
        rdma.wait()
        out_ref[...] += comm_ref[...]

    return pl.pallas_call(
        body,
        out_shape=jax.ShapeDtypeStruct((m, n), jnp.float32),
        in_specs=[
            pl.BlockSpec(memory_space=pltpu.VMEM),
            pl.BlockSpec(memory_space=pltpu.VMEM),
        ],
        out_specs=pl.BlockSpec(memory_space=pltpu.VMEM),
        scratch_shapes=[
            pltpu.VMEM((m, n), jnp.float32),
            pltpu.SemaphoreType.DMA,
            pltpu.SemaphoreType.DMA,
        ],
        compiler_params=pltpu.CompilerParams(collective_id=0),
    )(dy, W)


# device time: 53499 ns/iter; 1.3431x vs baseline; 1.3431x over previous
import jax
import jax.numpy as jnp
from jax import lax
from jax.experimental import pallas as pl
from jax.experimental.pallas import tpu as pltpu

C = 8


def kernel(dy, W):
    m, k = dy.shape
    n, k2 = W.shape
    assert k == k2
    h = n // 2
    r = m // C

    def body(dy_ref, w_ref, out_ref, pbuf, ybuf, red, xbuf,
             ysend, yrecv, xsend, xrecv):
        my_x = lax.axis_index("x")
        my_y = lax.axis_index("y")
        my_z = lax.axis_index("z")
        ypartner = (my_x, 1 - my_y, my_z)
        xpartner = (1 - my_x, my_y, my_z)

        barrier_sem = pltpu.get_barrier_semaphore()
        for nbr in (ypartner, xpartner):
            pl.semaphore_signal(
                barrier_sem, inc=1,
                device_id=nbr, device_id_type=pl.DeviceIdType.MESH,
            )
        pl.semaphore_wait(barrier_sem, 2)

        wx = w_ref[pl.ds(my_x * h, h), :]

        y_rdmas = []
        for c in range(C):
            pbuf[c] = lax.dot_general(
                dy_ref[pl.ds(c * r, r), :], wx,
                dimension_numbers=(((1,), (1,)), ((), ())),
                preferred_element_type=jnp.float32,
            )
            rdma = pltpu.make_async_remote_copy(
                src_ref=pbuf.at[c], dst_ref=ybuf.at[c],
                send_sem=ysend.at[c], recv_sem=yrecv.at[c],
                device_id=ypartner, device_id_type=pl.DeviceIdType.MESH,
            )
            rdma.start()
            y_rdmas.append(rdma)

        x_rdmas = []
        for c in range(C):
            y_rdmas[c].wait_recv()
            red[c] = pbuf[c] + ybuf[c]
            rdma = pltpu.make_async_remote_copy(
                src_ref=red.at[c], dst_ref=xbuf.at[c],
                send_sem=xsend.at[c], recv_sem=xrecv.at[c],
                device_id=xpartner, device_id_type=pl.DeviceIdType.MESH,
            )
            rdma.start()
            x_rdmas.append(rdma)
            out_ref[pl.ds(c * r, r), pl.ds(my_x * h, h)] = red[c]

        for c in range(C):
            x_rdmas[c].wait_recv()
            out_ref[pl.ds(c * r, r), pl.ds((1 - my_x) * h, h)] = xbuf[c]
            y_rdmas[c].wait_send()
            x_rdmas[c].wait_send()

    return pl.pallas_call(
        body,
        out_shape=jax.ShapeDtypeStruct((m, n), jnp.float32),
        in_specs=[
            pl.BlockSpec(memory_space=pltpu.VMEM),
            pl.BlockSpec(memory_space=pltpu.VMEM),
        ],
        out_specs=pl.BlockSpec(memory_space=pltpu.VMEM),
        scratch_shapes=[
            pltpu.VMEM((C, r, h), jnp.float32),
            pltpu.VMEM((C, r, h), jnp.float32),
            pltpu.VMEM((C, r, h), jnp.float32),
            pltpu.VMEM((C, r, h), jnp.float32),
            pltpu.SemaphoreType.DMA((C,)),
            pltpu.SemaphoreType.DMA((C,)),
            pltpu.SemaphoreType.DMA((C,)),
            pltpu.SemaphoreType.DMA((C,)),
        ],
        compiler_params=pltpu.CompilerParams(collective_id=0),
    )(dy, W)


# device time: 50362 ns/iter; 1.4268x vs baseline; 1.0623x over previous
import jax
import jax.numpy as jnp
from jax import lax
from jax.experimental import pallas as pl
from jax.experimental.pallas import tpu as pltpu

C = 8
DN = (((1,), (1,)), ((), ()))


def kernel(dy, W):
    m, k = dy.shape
    n, k2 = W.shape
    assert k == k2
    h = n // 2
    r = m // C

    def body(dy_hbm, w_hbm, out_ref, wxbuf, dybuf, pbuf, ybuf, red, xbuf,
             wx_sem, dy_sems, ysend, yrecv, xsend, xrecv):
        my_x = lax.axis_index("x")
        my_y = lax.axis_index("y")
        my_z = lax.axis_index("z")
        ypartner = (my_x, 1 - my_y, my_z)
        xpartner = (1 - my_x, my_y, my_z)

        wx_dma = pltpu.make_async_copy(
            w_hbm.at[pl.ds(my_x * h, h), :], wxbuf, wx_sem)
        wx_dma.start()
        dy_dmas = []
        for c in range(C):
            d = pltpu.make_async_copy(
                dy_hbm.at[pl.ds(c * r, r), :], dybuf.at[c], dy_sems.at[c])
            d.start()
            dy_dmas.append(d)

        barrier_sem = pltpu.get_barrier_semaphore()
        for nbr in (ypartner, xpartner):
            pl.semaphore_signal(
                barrier_sem, inc=1,
                device_id=nbr, device_id_type=pl.DeviceIdType.MESH,
            )
        pl.semaphore_wait(barrier_sem, 2)

        wx_dma.wait()
        y_rdmas = []
        for c in range(C):
            dy_dmas[c].wait()
            pbuf[c] = lax.dot_general(
                dybuf[c], wxbuf[...], DN,
                preferred_element_type=jnp.float32,
            )
            rdma = pltpu.make_async_remote_copy(
                src_ref=pbuf.at[c], dst_ref=ybuf.at[c],
                send_sem=ysend.at[c], recv_sem=yrecv.at[c],
                device_id=ypartner, device_id_type=pl.DeviceIdType.MESH,
            )
            rdma.start()
            y_rdmas.append(rdma)

        x_rdmas = []
        for c in range(C):
            y_rdmas[c].wait_recv()
            red[c] = pbuf[c] + ybuf[c]
            rdma = pltpu.make_async_remote_copy(
                src_ref=red.at[c], dst_ref=xbuf.at[c],
                send_sem=xsend.at[c], recv_sem=xrecv.at[c],
                device_id=xpartner, device_id_type=pl.DeviceIdType.MESH,
            )
            rdma.start()
            x_rdmas.append(rdma)
            out_ref[pl.ds(c * r, r), pl.ds(my_x * h, h)] = red[c]

        for c in range(C):
            x_rdmas[c].wait_recv()
            out_ref[pl.ds(c * r, r), pl.ds((1 - my_x) * h, h)] = xbuf[c]
            y_rdmas[c].wait_send()
            x_rdmas[c].wait_send()

    return pl.pallas_call(
        body,
        out_shape=jax.ShapeDtypeStruct((m, n), jnp.float32),
        in_specs=[
            pl.BlockSpec(memory_space=pltpu.MemorySpace.HBM),
            pl.BlockSpec(memory_space=pltpu.MemorySpace.HBM),
        ],
        out_specs=pl.BlockSpec(memory_space=pltpu.VMEM),
        scratch_shapes=[
            pltpu.VMEM((h, k), jnp.float32),
            pltpu.VMEM((C, r, k), jnp.float32),
            pltpu.VMEM((C, r, h), jnp.float32),
            pltpu.VMEM((C, r, h), jnp.float32),
            pltpu.VMEM((C, r, h), jnp.float32),
            pltpu.VMEM((C, r, h), jnp.float32),
            pltpu.SemaphoreType.DMA,
            pltpu.SemaphoreType.DMA((C,)),
            pltpu.SemaphoreType.DMA((C,)),
            pltpu.SemaphoreType.DMA((C,)),
            pltpu.SemaphoreType.DMA((C,)),
            pltpu.SemaphoreType.DMA((C,)),
        ],
        compiler_params=pltpu.CompilerParams(collective_id=0),
    )(dy, W)
